# Initial kernel scaffold; baseline (speedup 1.0000x reference)
import functools

import jax
import jax.numpy as jnp
from jax import lax
from jax.experimental import pallas as pl
from jax.experimental.pallas import tpu as pltpu
from jax.experimental.pallas import tpu_sc as plsc

N_NODES = 50000
D = 64
HALF = D // 2
NSUB = 16
CHUNK = 128
ZROWS = 521
ACC_ROWS = 50016
DUMP = N_NODES // NSUB
ZCHUNKS = 6


def _sc_aggregate(n_chunks):
    mesh = plsc.VectorSubcoreMesh(core_axis_name="c", subcore_axis_name="s")

    @functools.partial(
        pl.kernel,
        mesh=mesh,
        out_type=jax.ShapeDtypeStruct((2, N_NODES, HALF), jnp.float32),
        scratch_types=[
            pltpu.VMEM((n_chunks, CHUNK), jnp.int32),
            pltpu.VMEM((n_chunks, CHUNK), jnp.int32),
            pltpu.VMEM((CHUNK, HALF), jnp.float32),
            pltpu.VMEM((ZROWS, HALF), jnp.float32),
            pltpu.VMEM_SHARED((ACC_ROWS, HALF), jnp.float32),
            pltpu.SemaphoreType.DMA,
        ],
    )
    def agg_kernel(h_lo, h_hi, src_hbm, dst_hbm, out, src_v, dst_v, rows_v,
                   zbuf, acc, sem):
        c = lax.axis_index("c")
        s = lax.axis_index("s")
        zero16 = jnp.zeros((16,), jnp.float32)

        def zrow(i, carry):
            zbuf[i, pl.ds(0, 16)] = zero16
            zbuf[i, pl.ds(16, 16)] = zero16
            return carry

        lax.fori_loop(0, ZROWS, zrow, 0)

        def zcopy(k, carry):
            pltpu.sync_copy(zbuf, acc.at[pl.ds(s * (ZCHUNKS * ZROWS) + k * ZROWS,
                                               ZROWS)])
            return carry

        lax.fori_loop(0, ZCHUNKS, zcopy, 0)
        plsc.subcore_barrier()

        pltpu.sync_copy(src_hbm.at[s], src_v)
        pltpu.sync_copy(dst_hbm.at[s], dst_v)

        def step(j, carry):
            @pl.when(c == 0)
            def _():
                pltpu.async_copy(h_lo.at[src_v.at[j]], rows_v, sem).wait()

            @pl.when(c == 1)
            def _():
                pltpu.async_copy(h_hi.at[src_v.at[j]], rows_v, sem).wait()

            pltpu.sync_copy(rows_v, acc.at[dst_v.at[j]], add=True)
            return carry

        lax.fori_loop(0, n_chunks, step, 0)
        plsc.subcore_barrier()

        pltpu.sync_copy(acc.at[pl.ds(s * DUMP, DUMP)],
                        out.at[c, pl.ds(s * DUMP, DUMP)])

    return agg_kernel


def _mlp_body(agg_ref, hlo_ref, hhi_ref, w1_ref, b1_ref, w2_ref, b2_ref,
              olo_ref, ohi_ref, ofull_ref):
    rst_lo = agg_ref[0] + hlo_ref[...]
    rst_hi = agg_ref[1] + hhi_ref[...]
    rst = jnp.concatenate([rst_lo, rst_hi], axis=1)
    t = jnp.tanh(jnp.dot(rst, w1_ref[...], preferred_element_type=jnp.float32)
                 + b1_ref[...])
    o = jnp.dot(t, w2_ref[...], preferred_element_type=jnp.float32) + b2_ref[...]
    ofull_ref[...] = o
    olo_ref[...] = o[:, :HALF]
    ohi_ref[...] = o[:, HALF:]


def _mlp(bm):
    grid = (N_NODES // bm,)
    return pl.pallas_call(
        _mlp_body,
        grid=grid,
        in_specs=[
            pl.BlockSpec((2, bm, HALF), lambda i: (0, i, 0)),
            pl.BlockSpec((bm, HALF), lambda i: (i, 0)),
            pl.BlockSpec((bm, HALF), lambda i: (i, 0)),
            pl.BlockSpec((D, D), lambda i: (0, 0)),
            pl.BlockSpec((1, D), lambda i: (0, 0)),
            pl.BlockSpec((D, D), lambda i: (0, 0)),
            pl.BlockSpec((1, D), lambda i: (0, 0)),
        ],
        out_specs=[
            pl.BlockSpec((bm, HALF), lambda i: (i, 0)),
            pl.BlockSpec((bm, HALF), lambda i: (i, 0)),
            pl.BlockSpec((bm, D), lambda i: (i, 0)),
        ],
        out_shape=[
            jax.ShapeDtypeStruct((N_NODES, HALF), jnp.float32),
            jax.ShapeDtypeStruct((N_NODES, HALF), jnp.float32),
            jax.ShapeDtypeStruct((N_NODES, D), jnp.float32),
        ],
    )


def kernel(x, edge_index, W1, b1, W2, b2):
    n_edges = edge_index.shape[1]
    n_chunks = -(-n_edges // (NSUB * CHUNK))
    e_pad = NSUB * n_chunks * CHUNK

    src = edge_index[0].astype(jnp.int32)
    dst = edge_index[1].astype(jnp.int32)
    pad = e_pad - n_edges
    src_p = jnp.concatenate([src, jnp.zeros((pad,), jnp.int32)])
    dst_p = jnp.concatenate([dst, jnp.full((pad,), N_NODES, jnp.int32)])
    src_p = src_p.reshape(NSUB, n_chunks, CHUNK)
    dst_p = dst_p.reshape(NSUB, n_chunks, CHUNK)

    h_lo = x[:, :HALF]
    h_hi = x[:, HALF:]
    agg_fn = _sc_aggregate(n_chunks)
    mlp_fn = _mlp(2000)

    out = None
    for i in range(W1.shape[0]):
        agg = agg_fn(h_lo, h_hi, src_p, dst_p)
        h_lo, h_hi, out = mlp_fn(agg, h_lo, h_hi, W1[i], b1[i].reshape(1, D),
                                 W2[i], b2[i].reshape(1, D))
    return out


# speedup vs baseline: 3.4650x; 3.4650x over previous
import functools

import jax
import jax.numpy as jnp
from jax import lax
from jax.experimental import pallas as pl
from jax.experimental.pallas import tpu as pltpu
from jax.experimental.pallas import tpu_sc as plsc

N_NODES = 50000
D = 64
HALF = D // 2
NSUB = 16
CHUNK = 128
ZROWS = 200
ACC_ROWS = 51200
DUMP = ACC_ROWS // NSUB
ZCHUNKS = 16


def _sc_aggregate(n_chunks):
    mesh = plsc.VectorSubcoreMesh(core_axis_name="c", subcore_axis_name="s")

    @functools.partial(
        pl.kernel,
        mesh=mesh,
        out_type=jax.ShapeDtypeStruct((2, ACC_ROWS, HALF), jnp.float32),
        scratch_types=[
            pltpu.VMEM((CHUNK,), jnp.int32),
            pltpu.VMEM((CHUNK,), jnp.int32),
            pltpu.VMEM((CHUNK, HALF), jnp.float32),
            pltpu.VMEM((ZROWS, HALF), jnp.float32),
            pltpu.VMEM_SHARED((ACC_ROWS, HALF), jnp.float32),
            pltpu.SemaphoreType.DMA,
        ],
        compiler_params=pltpu.CompilerParams(use_tc_tiling_on_sc=False),
    )
    def agg_kernel(h_lo, h_hi, src_hbm, dst_hbm, out, src_v, dst_v, rows_v,
                   zbuf, acc, sem):
        c = lax.axis_index("c")
        s = lax.axis_index("s")
        zero16 = jnp.zeros((16,), jnp.float32)

        def zrow(i, carry):
            zbuf[i, pl.ds(0, 16)] = zero16
            zbuf[i, pl.ds(16, 16)] = zero16
            return carry

        lax.fori_loop(0, ZROWS, zrow, 0)

        def zcopy(k, carry):
            pltpu.sync_copy(zbuf, acc.at[pl.ds(s * (ZCHUNKS * ZROWS) + k * ZROWS,
                                               ZROWS)])
            return carry

        lax.fori_loop(0, ZCHUNKS, zcopy, 0)
        plsc.subcore_barrier()

        def step(j, carry):
            pltpu.sync_copy(src_hbm.at[s, j], src_v)
            pltpu.sync_copy(dst_hbm.at[s, j], dst_v)

            @pl.when(c == 0)
            def _():
                pltpu.async_copy(h_lo.at[src_v], rows_v, sem).wait()

            @pl.when(c == 1)
            def _():
                pltpu.async_copy(h_hi.at[src_v], rows_v, sem).wait()

            pltpu.sync_copy(rows_v, acc.at[dst_v], add=True)
            return carry

        lax.fori_loop(0, n_chunks, step, 0)
        plsc.subcore_barrier()

        pltpu.sync_copy(acc.at[pl.ds(s * DUMP, DUMP)],
                        out.at[c, pl.ds(s * DUMP, DUMP)])

    return agg_kernel


def _mlp_body(agg_ref, hlo_ref, hhi_ref, w1_ref, b1_ref, w2_ref, b2_ref,
              olo_ref, ohi_ref, ofull_ref):
    rst_lo = agg_ref[0] + hlo_ref[...]
    rst_hi = agg_ref[1] + hhi_ref[...]
    rst = jnp.concatenate([rst_lo, rst_hi], axis=1)
    t = jnp.tanh(jnp.dot(rst, w1_ref[...], preferred_element_type=jnp.float32)
                 + b1_ref[...])
    o = jnp.dot(t, w2_ref[...], preferred_element_type=jnp.float32) + b2_ref[...]
    ofull_ref[...] = o
    olo_ref[...] = o[:, :HALF]
    ohi_ref[...] = o[:, HALF:]


def _mlp(bm):
    grid = (N_NODES // bm,)
    return pl.pallas_call(
        _mlp_body,
        grid=grid,
        in_specs=[
            pl.BlockSpec((2, bm, HALF), lambda i: (0, i, 0)),
            pl.BlockSpec((bm, HALF), lambda i: (i, 0)),
            pl.BlockSpec((bm, HALF), lambda i: (i, 0)),
            pl.BlockSpec((D, D), lambda i: (0, 0)),
            pl.BlockSpec((1, D), lambda i: (0, 0)),
            pl.BlockSpec((D, D), lambda i: (0, 0)),
            pl.BlockSpec((1, D), lambda i: (0, 0)),
        ],
        out_specs=[
            pl.BlockSpec((bm, HALF), lambda i: (i, 0)),
            pl.BlockSpec((bm, HALF), lambda i: (i, 0)),
            pl.BlockSpec((bm, D), lambda i: (i, 0)),
        ],
        out_shape=[
            jax.ShapeDtypeStruct((N_NODES, HALF), jnp.float32),
            jax.ShapeDtypeStruct((N_NODES, HALF), jnp.float32),
            jax.ShapeDtypeStruct((N_NODES, D), jnp.float32),
        ],
    )


def kernel(x, edge_index, W1, b1, W2, b2):
    n_edges = edge_index.shape[1]
    n_chunks = -(-n_edges // (NSUB * CHUNK))
    e_pad = NSUB * n_chunks * CHUNK

    src = edge_index[0].astype(jnp.int32)
    dst = edge_index[1].astype(jnp.int32)
    pad = e_pad - n_edges
    src_p = jnp.concatenate([src, jnp.zeros((pad,), jnp.int32)])
    dst_p = jnp.concatenate([dst, jnp.full((pad,), N_NODES, jnp.int32)])
    src_p = src_p.reshape(NSUB, n_chunks, CHUNK)
    dst_p = dst_p.reshape(NSUB, n_chunks, CHUNK)

    h_lo = x[:, :HALF]
    h_hi = x[:, HALF:]
    agg_fn = _sc_aggregate(n_chunks)
    mlp_fn = _mlp(2000)

    out = None
    for i in range(W1.shape[0]):
        agg = agg_fn(h_lo, h_hi, src_p, dst_p)
        h_lo, h_hi, out = mlp_fn(agg, h_lo, h_hi, W1[i], b1[i].reshape(1, D),
                                 W2[i], b2[i].reshape(1, D))
    return out
